# baseline (speedup 1.0000x reference)
import jax
import jax.numpy as jnp
from jax import lax
from jax.experimental import pallas as pl
from jax.experimental.pallas import tpu as pltpu
from jax.experimental.pallas import tpu_sc as plsc

NC = 2
NS = 16
NW = NC * NS
LANES = 16
NBUF = 4
EPS = 1e-12


def _layernorm_rows(buf, posc, gamma_v, beta_v, n_rows, hid):
    nchunk = hid // LANES
    inv_hid = 1.0 / hid

    lane = lax.iota(jnp.int32, LANES)
    _dnums = lax.GatherDimensionNumbers(
        offset_dims=(), collapsed_slice_dims=(0,), start_index_map=(0,))

    def _shuffle(x, idx):
        return lax.gather(
            x, idx[:, None], _dnums, slice_sizes=(1,),
            mode=lax.GatherScatterMode.PROMISE_IN_BOUNDS)

    def _allsum(x):
        for k in (8, 4, 2, 1):
            x = x + _shuffle(x, lane ^ k)
        return x

    @pl.loop(0, n_rows)
    def _(r):
        acc_s = jnp.zeros((LANES,), jnp.float32)
        acc_q = jnp.zeros((LANES,), jnp.float32)
        for c in range(nchunk):
            sl = pl.ds(c * LANES, LANES)
            x = buf[r, sl] + posc[r, sl]
            buf[r, sl] = x
            acc_s = acc_s + x
            acc_q = acc_q + x * x
        mean_v = _allsum(acc_s) * inv_hid
        var_v = _allsum(acc_q) * inv_hid - mean_v * mean_v
        v = var_v + EPS
        seed = jnp.int32(0x5F3759DF) - (plsc.bitcast(v, jnp.int32) >> 1)
        y = plsc.bitcast(seed, jnp.float32)
        half = v * 0.5
        for _ in range(3):
            y = y * (1.5 - half * y * y)
        for c in range(nchunk):
            sl = pl.ds(c * LANES, LANES)
            a = gamma_v[sl] * y
            buf[r, sl] = (buf[r, sl] - mean_v) * a + beta_v[sl]


def _make_sc_kernel(B, S, hid):
    spw = S // NW
    nchunk = hid // LANES

    def body(ids_hbm, word_hbm, pos_hbm, type_hbm, gamma_hbm, beta_hbm,
             out_hbm, idx_all, posc, typ_v, gamma_v, beta_v, *rest):
        bufs = rest[:NBUF]
        gsems = rest[NBUF:2 * NBUF]
        ssems = rest[2 * NBUF:3 * NBUF]

        cid = lax.axis_index("c")
        sid = lax.axis_index("s")
        w = sid * NC + cid
        s0 = w * spw

        pltpu.sync_copy(ids_hbm.at[w], idx_all)
        pltpu.sync_copy(pos_hbm.at[pl.ds(s0, spw)], posc)
        pltpu.sync_copy(type_hbm.at[0], typ_v)
        pltpu.sync_copy(gamma_hbm, gamma_v)
        pltpu.sync_copy(beta_hbm, beta_v)

        @pl.loop(0, spw)
        def _(r):
            for c in range(nchunk):
                sl = pl.ds(c * LANES, LANES)
                posc[r, sl] = posc[r, sl] + typ_v[sl]

        def gather(b, j):
            return pltpu.make_async_copy(
                word_hbm.at[idx_all.at[b]], bufs[j], gsems[j])

        def scatter(b, j):
            return pltpu.make_async_copy(
                bufs[j], out_hbm.at[pl.ds(b * S + s0, spw)], ssems[j])

        gather(0, 0).start()
        gather(1, 1).start()

        @pl.loop(0, B, step=NBUF)
        def _(b0):
            for jj in range(NBUF):
                b = b0 + jj
                g = b + 2
                jg = (jj + 2) % NBUF

                @pl.when(g < B)
                def _():
                    @pl.when(g >= NBUF)
                    def _():
                        scatter(g - NBUF, jg).wait()
                    gather(g, jg).start()

                gather(b, jj).wait()
                _layernorm_rows(bufs[jj], posc, gamma_v, beta_v, spw, hid)
                scatter(b, jj).start()

        for jj in range(NBUF):
            scatter(B - NBUF + jj, jj).wait()

    mesh = plsc.VectorSubcoreMesh(core_axis_name="c", subcore_axis_name="s")
    scratch = [
        pltpu.VMEM((B, spw), jnp.int32),
        pltpu.VMEM((spw, hid), jnp.float32),
        pltpu.VMEM((hid,), jnp.float32),
        pltpu.VMEM((hid,), jnp.float32),
        pltpu.VMEM((hid,), jnp.float32),
    ]
    scratch += [pltpu.VMEM((spw, hid), jnp.float32) for _ in range(NBUF)]
    scratch += [pltpu.SemaphoreType.DMA for _ in range(2 * NBUF)]

    return pl.kernel(
        body,
        out_type=jax.ShapeDtypeStruct((B * S, hid), jnp.float32),
        mesh=mesh,
        scratch_types=scratch,
        compiler_params=pltpu.CompilerParams(needs_layout_passes=False),
    )


@jax.jit
def _run(input_ids, word_table, pos_table, type_table, gamma, beta):
    B, S = input_ids.shape
    hid = word_table.shape[1]
    spw = S // NW
    ids = input_ids.astype(jnp.int32)
    ids_r = ids.reshape(B, NW, spw).transpose(1, 0, 2)
    fn = _make_sc_kernel(B, S, hid)
    out = fn(ids_r, word_table, pos_table, type_table, gamma, beta)
    return out.reshape(B, S, hid)


def kernel(input_ids, word_table, pos_table, type_table, gamma, beta):
    return _run(input_ids, word_table, pos_table, type_table, gamma, beta)
